# baseline (speedup 1.0000x reference)
import jax
import jax.numpy as jnp
from jax.experimental import pallas as pl


def _add_kernel(x_ref, pos_ref, out_ref):
    out_ref[...] = x_ref[...] + pos_ref[...][None]


def kernel(X, pos_embedding):
    B, L, D = X.shape
    Db = 512
    grid = (D // Db, B)
    return pl.pallas_call(
        _add_kernel,
        grid=grid,
        in_specs=[
            pl.BlockSpec((1, L, Db), lambda d, b: (b, 0, d)),
            pl.BlockSpec((L, Db), lambda d, b: (0, d)),
        ],
        out_specs=pl.BlockSpec((1, L, Db), lambda d, b: (b, 0, d)),
        out_shape=jax.ShapeDtypeStruct((B, L, D), X.dtype),
    )(X, pos_embedding)


# speedup vs baseline: 1.1027x; 1.1027x over previous
import jax
import jax.numpy as jnp
from jax.experimental import pallas as pl


def _add_kernel(x_ref, pos_ref, out_ref):
    out_ref[...] = x_ref[...] + pos_ref[...]


def kernel(X, pos_embedding):
    B, L, D = X.shape
    out = pl.pallas_call(
        _add_kernel,
        grid=(B,),
        in_specs=[
            pl.BlockSpec((L, D), lambda b: (b, 0)),
            pl.BlockSpec((L, D), lambda b: (0, 0)),
        ],
        out_specs=pl.BlockSpec((L, D), lambda b: (b, 0)),
        out_shape=jax.ShapeDtypeStruct((B * L, D), X.dtype),
    )(X.reshape(B * L, D), pos_embedding)
    return out.reshape(B, L, D)
